# baseline (speedup 1.0000x reference)
import functools
import math

import jax
import jax.numpy as jnp
from jax.experimental import pallas as pl


def _body(w_ref, o_ref, *, rows, scale, half):
    shift = (pl.program_id(0) * rows).astype(jnp.float32)
    w = w_ref[...]
    ws = w[:, :half]
    wc = w[:, half:]
    j = jax.lax.broadcasted_iota(jnp.int32, (1, half), 1).astype(jnp.float32)
    ang = shift * jnp.exp(j * (-scale))
    c = jnp.cos(ang)
    s = jnp.sin(ang)
    o_ref[:, :half] = ws * c + wc * s
    o_ref[:, half:] = wc * c - ws * s


def kernel(input, weights):
    _, dim = weights.shape
    half = dim // 2
    seq_len = input.shape[1]
    scale = math.log(10000.0) / (half - 1)
    rows = 128
    out = pl.pallas_call(
        functools.partial(_body, rows=rows, scale=scale, half=half),
        grid=(seq_len // rows,),
        in_specs=[pl.BlockSpec((rows, dim), lambda i: (0, 0))],
        out_specs=pl.BlockSpec((rows, dim), lambda i: (i, 0)),
        out_shape=jax.ShapeDtypeStruct((seq_len, dim), jnp.float32),
    )(weights)
    return jax.lax.stop_gradient(out)


# speedup vs baseline: 1.1680x; 1.1680x over previous
import functools
import math

import jax
import jax.numpy as jnp
from jax.experimental import pallas as pl


def _body(w_ref, o_ref, *, rows, scale, half):
    shift = (pl.program_id(0) * rows).astype(jnp.float32)
    w = w_ref[...]
    ws = w[:, :half]
    wc = w[:, half:]
    j = jax.lax.broadcasted_iota(jnp.int32, (1, half), 1).astype(jnp.float32)
    ang = shift * jnp.exp(j * (-scale))
    c = jnp.cos(ang)
    s = jnp.sin(ang)
    o_ref[:, :half] = ws * c + wc * s
    o_ref[:, half:] = wc * c - ws * s


def kernel(input, weights):
    _, dim = weights.shape
    half = dim // 2
    seq_len = input.shape[1]
    scale = math.log(10000.0) / (half - 1)
    rows = 256
    out = pl.pallas_call(
        functools.partial(_body, rows=rows, scale=scale, half=half),
        grid=(seq_len // rows,),
        in_specs=[pl.BlockSpec((rows, dim), lambda i: (0, 0))],
        out_specs=pl.BlockSpec((rows, dim), lambda i: (i, 0)),
        out_shape=jax.ShapeDtypeStruct((seq_len, dim), jnp.float32),
    )(weights)
    return jax.lax.stop_gradient(out)


# speedup vs baseline: 1.2065x; 1.0329x over previous
import functools
import math

import jax
import jax.numpy as jnp
from jax.experimental import pallas as pl
from jax.experimental.pallas import tpu as pltpu


def _body(w_ref, o_ref, c_ref, s_ref, *, rows, scale, half, nsteps):
    i = pl.program_id(0)

    @pl.when(i == 0)
    def _():
        k = jax.lax.broadcasted_iota(jnp.int32, (nsteps, half), 0).astype(jnp.float32)
        j = jax.lax.broadcasted_iota(jnp.int32, (nsteps, half), 1).astype(jnp.float32)
        ang = (k * float(rows)) * jnp.exp(j * (-scale))
        c_ref[...] = jnp.cos(ang)
        s_ref[...] = jnp.sin(ang)

    w = w_ref[...]
    ws = w[:, :half]
    wc = w[:, half:]
    c = c_ref[pl.ds(i, 1), :]
    s = s_ref[pl.ds(i, 1), :]
    o_ref[:, :half] = ws * c + wc * s
    o_ref[:, half:] = wc * c - ws * s


def kernel(input, weights):
    _, dim = weights.shape
    half = dim // 2
    seq_len = input.shape[1]
    scale = math.log(10000.0) / (half - 1)
    rows = 256
    nsteps = seq_len // rows
    out = pl.pallas_call(
        functools.partial(_body, rows=rows, scale=scale, half=half, nsteps=nsteps),
        grid=(nsteps,),
        in_specs=[pl.BlockSpec((rows, dim), lambda i: (0, 0))],
        out_specs=pl.BlockSpec((rows, dim), lambda i: (i, 0)),
        out_shape=jax.ShapeDtypeStruct((seq_len, dim), jnp.float32),
        scratch_shapes=[
            pltpu.VMEM((nsteps, half), jnp.float32),
            pltpu.VMEM((nsteps, half), jnp.float32),
        ],
    )(weights)
    return jax.lax.stop_gradient(out)
